# baseline (speedup 1.0000x reference)
import functools

import jax
import jax.numpy as jnp
from jax.experimental import pallas as pl
from jax.experimental.pallas import tpu as pltpu

_B = 16384
_D = 64
_W = 128
_R = _B * _D // _W
_BLK = 1024
_GRID = _R // _BLK


def _body(gu_ref, gi_ref, m2_ref, xui_ref, gamu_ref, gami_ref):
    gu = gu_ref[...]
    gi = gi_ref[...]
    gamu_ref[...] = gu
    gami_ref[...] = gi
    pair = jax.lax.dot_general(
        gu * gi, m2_ref[...], (((1,), (0,)), ((), ())),
        preferred_element_type=jnp.float32,
        precision=jax.lax.Precision.HIGHEST)
    xui_ref[...] = pair.T


@jax.jit
def _uuii_tc(gu2, gi2):
    m2 = jnp.repeat(jnp.eye(2, dtype=jnp.float32), _D, axis=0)
    return pl.pallas_call(
        _body,
        grid=(_GRID,),
        in_specs=[
            pl.BlockSpec((_BLK, _W), lambda i: (i, 0)),
            pl.BlockSpec((_BLK, _W), lambda i: (i, 0)),
            pl.BlockSpec((_W, 2), lambda i: (0, 0)),
        ],
        out_specs=[
            pl.BlockSpec((2, _BLK), lambda i: (0, i)),
            pl.BlockSpec((_BLK, _W), lambda i: (i, 0)),
            pl.BlockSpec((_BLK, _W), lambda i: (i, 0)),
        ],
        out_shape=[
            jax.ShapeDtypeStruct((2, _R), jnp.float32),
            jax.ShapeDtypeStruct((_R, _W), jnp.float32),
            jax.ShapeDtypeStruct((_R, _W), jnp.float32),
        ],
        compiler_params=pltpu.CompilerParams(
            dimension_semantics=("arbitrary",),
        ),
    )(gu2, gi2, m2)


def kernel(gu, gi):
    xuit, gamu2, gami2 = _uuii_tc(gu.reshape(_R, _W), gi.reshape(_R, _W))
    return (xuit.T.reshape(_B), gamu2.reshape(_B, _D), gami2.reshape(_B, _D))


# speedup vs baseline: 1.6755x; 1.6755x over previous
import jax
import jax.numpy as jnp
from jax.experimental import pallas as pl
from jax.experimental.pallas import tpu as pltpu

_B = 16384
_D = 64
_BLK = 2048
_GRID = _B // _BLK


def _body(gu_ref, gi_ref, gamu_ref, gami_ref):
    gamu_ref[...] = gu_ref[...]
    gami_ref[...] = gi_ref[...]


@jax.jit
def _copies(gu, gi):
    return pl.pallas_call(
        _body,
        grid=(_GRID,),
        in_specs=[
            pl.BlockSpec((_BLK, _D), lambda i: (i, 0)),
            pl.BlockSpec((_BLK, _D), lambda i: (i, 0)),
        ],
        out_specs=[
            pl.BlockSpec((_BLK, _D), lambda i: (i, 0)),
            pl.BlockSpec((_BLK, _D), lambda i: (i, 0)),
        ],
        out_shape=[
            jax.ShapeDtypeStruct((_B, _D), jnp.float32),
            jax.ShapeDtypeStruct((_B, _D), jnp.float32),
        ],
        compiler_params=pltpu.CompilerParams(
            dimension_semantics=("arbitrary",),
        ),
    )(gu, gi)


def kernel(gu, gi):
    gamma_u, gamma_i = _copies(gu, gi)
    xui = jnp.sum(gu * gi, axis=1)
    return (xui, gamma_u, gamma_i)


# speedup vs baseline: 1.7272x; 1.0309x over previous
import jax
import jax.numpy as jnp
from jax.experimental import pallas as pl
from jax.experimental.pallas import tpu as pltpu

_B = 16384
_D = 64
_BLK = 16384
_GRID = _B // _BLK


def _body(gu_ref, gi_ref, gamu_ref, gami_ref):
    gamu_ref[...] = gu_ref[...]
    gami_ref[...] = gi_ref[...]


@jax.jit
def _copies(gu, gi):
    return pl.pallas_call(
        _body,
        grid=(_GRID,),
        in_specs=[
            pl.BlockSpec((_BLK, _D), lambda i: (i, 0)),
            pl.BlockSpec((_BLK, _D), lambda i: (i, 0)),
        ],
        out_specs=[
            pl.BlockSpec((_BLK, _D), lambda i: (i, 0)),
            pl.BlockSpec((_BLK, _D), lambda i: (i, 0)),
        ],
        out_shape=[
            jax.ShapeDtypeStruct((_B, _D), jnp.float32),
            jax.ShapeDtypeStruct((_B, _D), jnp.float32),
        ],
        compiler_params=pltpu.CompilerParams(
            dimension_semantics=("arbitrary",),
        ),
    )(gu, gi)


def kernel(gu, gi):
    gamma_u, gamma_i = _copies(gu, gi)
    xui = jnp.sum(gu * gi, axis=1)
    return (xui, gamma_u, gamma_i)


# speedup vs baseline: 1.8595x; 1.0766x over previous
import functools

import jax
import jax.numpy as jnp
from jax.experimental import pallas as pl
from jax.experimental.pallas import tpu as pltpu

_B = 16384
_D = 64
_NCH = 8
_CH = _B // _NCH
_NBUF = 4


def _body(gu_ref, gi_ref, xui_ref, gamu_ref, gami_ref,
          u_buf, i_buf, xacc, sin_u, sin_i, sout_u, sout_i, sx):

    def cp_in(c, b):
        sl = pl.ds(c * _CH, _CH)
        return (pltpu.make_async_copy(gu_ref.at[sl], u_buf.at[b], sin_u.at[b]),
                pltpu.make_async_copy(gi_ref.at[sl], i_buf.at[b], sin_i.at[b]))

    def cp_out(c, b):
        sl = pl.ds(c * _CH, _CH)
        return (pltpu.make_async_copy(u_buf.at[b], gamu_ref.at[sl], sout_u.at[b]),
                pltpu.make_async_copy(i_buf.at[b], gami_ref.at[sl], sout_i.at[b]))

    for c in range(3):
        for cp in cp_in(c, c % _NBUF):
            cp.start()

    for c in range(_NCH):
        b = c % _NBUF
        for cp in cp_in(c, b):
            cp.wait()
        xacc[pl.ds(c * _CH, _CH)] = jnp.sum(u_buf[b] * i_buf[b], axis=1)
        for cp in cp_out(c, b):
            cp.start()
        nxt = c + 3
        if nxt < _NCH:
            nb = nxt % _NBUF
            prev = nxt - _NBUF
            if prev >= 0:
                for cp in cp_out(prev, nb):
                    cp.wait()
            for cp in cp_in(nxt, nb):
                cp.start()

    for c in range(_NCH - _NBUF, _NCH):
        for cp in cp_out(c, c % _NBUF):
            cp.wait()

    xcp = pltpu.make_async_copy(xacc, xui_ref, sx)
    xcp.start()
    xcp.wait()


@jax.jit
def _uuii_tc(gu, gi):
    return pl.pallas_call(
        _body,
        in_specs=[
            pl.BlockSpec(memory_space=pl.MemorySpace.ANY),
            pl.BlockSpec(memory_space=pl.MemorySpace.ANY),
        ],
        out_specs=[
            pl.BlockSpec(memory_space=pl.MemorySpace.ANY),
            pl.BlockSpec(memory_space=pl.MemorySpace.ANY),
            pl.BlockSpec(memory_space=pl.MemorySpace.ANY),
        ],
        out_shape=[
            jax.ShapeDtypeStruct((_B,), jnp.float32),
            jax.ShapeDtypeStruct((_B, _D), jnp.float32),
            jax.ShapeDtypeStruct((_B, _D), jnp.float32),
        ],
        scratch_shapes=[
            pltpu.VMEM((_NBUF, _CH, _D), jnp.float32),
            pltpu.VMEM((_NBUF, _CH, _D), jnp.float32),
            pltpu.VMEM((_B,), jnp.float32),
            pltpu.SemaphoreType.DMA((_NBUF,)),
            pltpu.SemaphoreType.DMA((_NBUF,)),
            pltpu.SemaphoreType.DMA((_NBUF,)),
            pltpu.SemaphoreType.DMA((_NBUF,)),
            pltpu.SemaphoreType.DMA,
        ],
    )(gu, gi)


def kernel(gu, gi):
    xui, gamma_u, gamma_i = _uuii_tc(gu, gi)
    return (xui, gamma_u, gamma_i)
